# baseline (speedup 1.0000x reference)
import jax
import jax.numpy as jnp
from jax.experimental import pallas as pl


def _gcn_conv(x, src, dst, W, b):
    n = x.shape[0]
    loop = jnp.arange(n, dtype=src.dtype)
    s = jnp.concatenate([src, loop])
    d = jnp.concatenate([dst, loop])
    deg = jnp.zeros((n,), dtype=x.dtype).at[d].add(1.0)
    dinv = jnp.where(deg > 0, 1.0 / jnp.sqrt(deg), 0.0)
    norm = dinv[s] * dinv[d]
    h = x @ W
    msg = h[s] * norm[:, None]
    out = jnp.zeros((n, W.shape[1]), dtype=x.dtype).at[d].add(msg)
    return out + b


def kernel(x, edge_index, W1, b1, W2, b2, Wc, bc):
    src = edge_index[0]
    dst = edge_index[1]
    h1 = jax.nn.relu(_gcn_conv(x, src, dst, W1, b1))
    h2 = jax.nn.relu(_gcn_conv(h1, src, dst, W2, b2))
    combined = jnp.concatenate([h2, x], axis=1)
    return combined @ Wc + bc


# speedup vs baseline: 11.8709x; 11.8709x over previous
import functools

import jax
import jax.numpy as jnp
from jax import lax
from jax.experimental import pallas as pl
from jax.experimental.pallas import tpu as pltpu
from jax.experimental.pallas import tpu_sc as plsc

N = 10000
D = 128
E = 320000
NC = 2
NS = 16
NW = NC * NS
E_W = E // NW
CH = 80
NCH = E_W // CH
N_PAD = 10240
ROWS_W = N_PAD // NS

_mesh = plsc.VectorSubcoreMesh(core_axis_name="c", subcore_axis_name="s")


@functools.partial(
    pl.kernel,
    out_type=jax.ShapeDtypeStruct((NC, 1, N_PAD), jnp.float32),
    mesh=_mesh,
    scratch_types=[
        pltpu.VMEM((CH,), jnp.int32),
        pltpu.VMEM((CH,), jnp.float32),
        pltpu.VMEM((N_PAD // NS,), jnp.float32),
        pltpu.VMEM_SHARED((N_PAD,), jnp.float32),
    ],
)
def _deg_partials(dst_hbm, deg_out, idxv, onesv, zb, acc):
    c = lax.axis_index("c")
    s = lax.axis_index("s")
    wid = s * NC + c
    ebase = wid * E_W
    stripe = N_PAD // NS
    for i in range(CH // 16):
        onesv[pl.ds(i * 16, 16)] = jnp.full((16,), 1.0, jnp.float32)
    for i in range(stripe // 16):
        zb[pl.ds(i * 16, 16)] = jnp.zeros((16,), jnp.float32)
    pltpu.sync_copy(zb, acc.at[pl.ds(s * stripe, stripe)])
    plsc.subcore_barrier()

    def ebody(i, carry):
        pltpu.sync_copy(dst_hbm.at[pl.ds(ebase + i * CH, CH)], idxv)
        pltpu.sync_copy(onesv, acc.at[idxv], add=True)
        return carry

    lax.fori_loop(0, NCH, ebody, 0)
    plsc.subcore_barrier()
    pltpu.sync_copy(acc.at[pl.ds(s * stripe, stripe)],
                    deg_out.at[c, 0, pl.ds(s * stripe, stripe)])


@functools.partial(
    pl.kernel,
    out_type=jax.ShapeDtypeStruct((NC, N_PAD, D), jnp.float32),
    mesh=_mesh,
    scratch_types=[
        pltpu.VMEM((CH,), jnp.int32),
        pltpu.VMEM((CH,), jnp.int32),
        pltpu.VMEM((CH, D), jnp.float32),
        pltpu.VMEM((32, D), jnp.float32),
        pltpu.VMEM_SHARED((N_PAD, D), jnp.float32),
        pltpu.SemaphoreType.DMA,
    ],
)
def _agg_partials(hs_hbm, src_hbm, dst_hbm, out_hbm, srcv, dstv, rows, zb, acc, sem):
    c = lax.axis_index("c")
    s = lax.axis_index("s")
    wid = s * NC + c
    ebase = wid * E_W
    rbase = s * ROWS_W
    for i in range(32):
        for j in range(D // 16):
            zb[i, pl.ds(j * 16, 16)] = jnp.zeros((16,), jnp.float32)

    def zbody(i, carry):
        pltpu.sync_copy(zb, acc.at[pl.ds(rbase + i * 32, 32)])
        return carry

    lax.fori_loop(0, ROWS_W // 32, zbody, 0)
    plsc.subcore_barrier()

    def ebody(i, carry):
        eb = ebase + i * CH
        pltpu.sync_copy(src_hbm.at[pl.ds(eb, CH)], srcv)
        pltpu.sync_copy(dst_hbm.at[pl.ds(eb, CH)], dstv)
        pltpu.async_copy(hs_hbm.at[srcv], rows, sem).wait()
        pltpu.sync_copy(rows, acc.at[dstv], add=True)
        return carry

    lax.fori_loop(0, NCH, ebody, 0)
    plsc.subcore_barrier()
    pltpu.sync_copy(acc.at[pl.ds(rbase, ROWS_W)],
                    out_hbm.at[c, pl.ds(rbase, ROWS_W)])


_BLK = 2000


def _mm2_body(x_ref, wa_ref, wb_ref, oa_ref, ob_ref):
    x = x_ref[...]
    oa_ref[...] = jnp.dot(x, wa_ref[...], preferred_element_type=jnp.float32)
    ob_ref[...] = jnp.dot(x, wb_ref[...], preferred_element_type=jnp.float32)


def _prescale_body(h_ref, d0_ref, d1_ref, o_ref):
    dinv = lax.rsqrt(d0_ref[...] + d1_ref[...] + 1.0)
    o_ref[...] = h_ref[...] * dinv


def _combine_body(p0_ref, p1_ref, hs_ref, d0_ref, d1_ref, b_ref, w_ref, o_ref):
    dinv = lax.rsqrt(d0_ref[...] + d1_ref[...] + 1.0)
    z = dinv * (p0_ref[...] + p1_ref[...] + hs_ref[...]) + b_ref[...]
    z = jnp.maximum(z, 0.0)
    o_ref[...] = jnp.dot(z, w_ref[...], preferred_element_type=jnp.float32) * dinv


def _final_body(p0_ref, p1_ref, hs_ref, d0_ref, d1_ref, b_ref, w_ref, xc_ref,
                bc_ref, o_ref):
    dinv = lax.rsqrt(d0_ref[...] + d1_ref[...] + 1.0)
    z = dinv * (p0_ref[...] + p1_ref[...] + hs_ref[...]) + b_ref[...]
    z = jnp.maximum(z, 0.0)
    o_ref[...] = (jnp.dot(z, w_ref[...], preferred_element_type=jnp.float32)
                  + xc_ref[...] + bc_ref[...])


def _row_spec(w):
    return pl.BlockSpec((_BLK, w), lambda i: (i, 0))


def _full_spec(r, w):
    return pl.BlockSpec((r, w), lambda i: (0, 0))


_G = N // _BLK

_mm2 = pl.pallas_call(
    _mm2_body,
    grid=(_G,),
    in_specs=[_row_spec(D), _full_spec(D, D), _full_spec(D, 64)],
    out_specs=[_row_spec(D), _row_spec(64)],
    out_shape=[jax.ShapeDtypeStruct((N, D), jnp.float32),
               jax.ShapeDtypeStruct((N, 64), jnp.float32)],
)

_prescale = pl.pallas_call(
    _prescale_body,
    grid=(_G,),
    in_specs=[_row_spec(D), _row_spec(1), _row_spec(1)],
    out_specs=_row_spec(D),
    out_shape=jax.ShapeDtypeStruct((N, D), jnp.float32),
)

_combine = pl.pallas_call(
    _combine_body,
    grid=(_G,),
    in_specs=[_row_spec(D), _row_spec(D), _row_spec(D), _row_spec(1),
              _row_spec(1), _full_spec(1, D), _full_spec(D, D)],
    out_specs=_row_spec(D),
    out_shape=jax.ShapeDtypeStruct((N, D), jnp.float32),
)

_final = pl.pallas_call(
    _final_body,
    grid=(_G,),
    in_specs=[_row_spec(D), _row_spec(D), _row_spec(D), _row_spec(1),
              _row_spec(1), _full_spec(1, D), _full_spec(D, 64),
              _row_spec(64), _full_spec(1, 64)],
    out_specs=_row_spec(64),
    out_shape=jax.ShapeDtypeStruct((N, 64), jnp.float32),
)


def kernel(x, edge_index, W1, b1, W2, b2, Wc, bc):
    src = edge_index[0].astype(jnp.int32)
    dst = edge_index[1].astype(jnp.int32)

    degp = _deg_partials(dst)
    h1, xc = _mm2(x, W1, Wc[D:])
    d0 = degp[0, 0, :N].reshape(N, 1)
    d1 = degp[1, 0, :N].reshape(N, 1)

    hs1 = _prescale(h1, d0, d1)
    p1 = _agg_partials(hs1, src, dst)
    hs2 = _combine(p1[0], p1[1], hs1, d0, d1,
                   b1.reshape(1, D), W2)
    p2 = _agg_partials(hs2, src, dst)
    out = _final(p2[0], p2[1], hs2, d0, d1,
                 b2.reshape(1, D), Wc[:D], xc,
                 bc.reshape(1, 64))
    return out


# speedup vs baseline: 26.5585x; 2.2373x over previous
import functools

import jax
import jax.numpy as jnp
from jax import lax
from jax.experimental import pallas as pl
from jax.experimental.pallas import tpu as pltpu
from jax.experimental.pallas import tpu_sc as plsc

N = 10000
D = 128
E = 320000
NC = 2
NS = 16
NW = NC * NS
E_W = E // NW
N_PAD = 10240
ROWS_W = N_PAD // NS
E_WP = 10240
CHP = 128
NCHP = E_WP // CHP
BLK_CH = 16
E_BLK = BLK_CH * CHP
NBLK = E_WP // E_BLK

_mesh = plsc.VectorSubcoreMesh(core_axis_name="c", subcore_axis_name="s")


@functools.partial(
    pl.kernel,
    out_type=jax.ShapeDtypeStruct((NC, 1, N_PAD), jnp.float32),
    mesh=_mesh,
    scratch_types=[
        pltpu.VMEM((NCHP, CHP), jnp.int32),
        pltpu.VMEM((CHP,), jnp.float32),
        pltpu.VMEM((ROWS_W,), jnp.float32),
        pltpu.VMEM_SHARED((N_PAD,), jnp.float32),
    ],
)
def _deg_partials(dst_hbm, deg_out, didx, onesv, zb, acc):
    c = lax.axis_index("c")
    s = lax.axis_index("s")
    wid = s * NC + c
    for i in range(CHP // 16):
        onesv[pl.ds(i * 16, 16)] = jnp.full((16,), 1.0, jnp.float32)
    for i in range(ROWS_W // 16):
        zb[pl.ds(i * 16, 16)] = jnp.zeros((16,), jnp.float32)
    pltpu.sync_copy(dst_hbm.at[wid], didx)
    pltpu.sync_copy(zb, acc.at[pl.ds(s * ROWS_W, ROWS_W)])
    plsc.subcore_barrier()

    def ebody(i, carry):
        pltpu.sync_copy(onesv, acc.at[didx.at[i]], add=True)
        return carry

    lax.fori_loop(0, NCHP, ebody, 0)
    plsc.subcore_barrier()
    pltpu.sync_copy(acc.at[pl.ds(s * ROWS_W, ROWS_W)],
                    deg_out.at[c, 0, pl.ds(s * ROWS_W, ROWS_W)])


@functools.partial(
    pl.kernel,
    out_type=jax.ShapeDtypeStruct((NC, N_PAD, D), jnp.float32),
    mesh=_mesh,
    scratch_types=[
        pltpu.VMEM((E_BLK,), jnp.int32),
        pltpu.VMEM((BLK_CH, CHP), jnp.int32),
        pltpu.VMEM((CHP, D), jnp.float32),
        pltpu.VMEM((CHP, D), jnp.float32),
        pltpu.VMEM_SHARED((N_PAD, D), jnp.float32),
        pltpu.SemaphoreType.DMA,
        pltpu.SemaphoreType.DMA,
    ],
)
def _agg_partials(hs_hbm, src_hbm, dst_hbm, out_hbm,
                  sidx, didx, rows0, rows1, acc, sem0, sem1):
    c = lax.axis_index("c")
    s = lax.axis_index("s")
    wid = s * NC + c
    rbase = s * ROWS_W

    for i in range(32):
        for j in range(D // 16):
            rows1[i, pl.ds(j * 16, 16)] = jnp.zeros((16,), jnp.float32)

    def zbody(i, carry):
        pltpu.sync_copy(rows1.at[pl.ds(0, 32)], acc.at[pl.ds(rbase + i * 32, 32)])
        return carry

    lax.fori_loop(0, ROWS_W // 32, zbody, 0)
    plsc.subcore_barrier()

    def blk_body(blk, carry):
        pltpu.sync_copy(src_hbm.at[wid, 0, pl.ds(blk * E_BLK, E_BLK)], sidx)
        pltpu.sync_copy(dst_hbm.at[wid, pl.ds(blk * BLK_CH, BLK_CH)], didx)
        pltpu.async_copy(hs_hbm.at[sidx.at[pl.ds(0, CHP)]], rows0, sem0)

        def body(i, carry2):
            b0 = 2 * i
            b1 = 2 * i + 1
            bn = jnp.minimum(b0 + 2, BLK_CH - 1)
            pltpu.async_copy(hs_hbm.at[sidx.at[pl.ds(b1 * CHP, CHP)]], rows1, sem1)
            pltpu.make_async_copy(hs_hbm.at[sidx.at[pl.ds(b0 * CHP, CHP)]],
                                  rows0, sem0).wait()
            pltpu.sync_copy(rows0, acc.at[didx.at[b0]], add=True)
            pltpu.async_copy(hs_hbm.at[sidx.at[pl.ds(bn * CHP, CHP)]], rows0, sem0)
            pltpu.make_async_copy(hs_hbm.at[sidx.at[pl.ds(b1 * CHP, CHP)]],
                                  rows1, sem1).wait()
            pltpu.sync_copy(rows1, acc.at[didx.at[b1]], add=True)
            return carry2

        lax.fori_loop(0, BLK_CH // 2, body, 0)
        pltpu.make_async_copy(hs_hbm.at[sidx.at[pl.ds(0, CHP)]], rows0, sem0).wait()
        return carry

    lax.fori_loop(0, NBLK, blk_body, 0)
    plsc.subcore_barrier()
    pltpu.sync_copy(acc.at[pl.ds(rbase, ROWS_W)],
                    out_hbm.at[c, pl.ds(rbase, ROWS_W)])


_BLK = 2000


def _mm2_body(x_ref, wa_ref, wb_ref, oa_ref, ob_ref):
    x = x_ref[...]
    oa_ref[...] = jnp.dot(x, wa_ref[...], preferred_element_type=jnp.float32)
    ob_ref[...] = jnp.dot(x, wb_ref[...], preferred_element_type=jnp.float32)


def _prescale_body(h_ref, d0_ref, d1_ref, o_ref):
    dinv = lax.rsqrt(d0_ref[...] + d1_ref[...] + 1.0)
    o_ref[...] = h_ref[...] * dinv


def _combine_body(p0_ref, p1_ref, hs_ref, d0_ref, d1_ref, b_ref, w_ref, o_ref):
    dinv = lax.rsqrt(d0_ref[...] + d1_ref[...] + 1.0)
    z = dinv * (p0_ref[...] + p1_ref[...] + hs_ref[...]) + b_ref[...]
    z = jnp.maximum(z, 0.0)
    o_ref[...] = jnp.dot(z, w_ref[...], preferred_element_type=jnp.float32) * dinv


def _final_body(p0_ref, p1_ref, hs_ref, d0_ref, d1_ref, b_ref, w_ref, xc_ref,
                bc_ref, o_ref):
    dinv = lax.rsqrt(d0_ref[...] + d1_ref[...] + 1.0)
    z = dinv * (p0_ref[...] + p1_ref[...] + hs_ref[...]) + b_ref[...]
    z = jnp.maximum(z, 0.0)
    o_ref[...] = (jnp.dot(z, w_ref[...], preferred_element_type=jnp.float32)
                  + xc_ref[...] + bc_ref[...])


def _row_spec(w):
    return pl.BlockSpec((_BLK, w), lambda i: (i, 0))


def _full_spec(r, w):
    return pl.BlockSpec((r, w), lambda i: (0, 0))


_G = N // _BLK

_mm2 = pl.pallas_call(
    _mm2_body,
    grid=(_G,),
    in_specs=[_row_spec(D), _full_spec(D, D), _full_spec(D, 64)],
    out_specs=[_row_spec(D), _row_spec(64)],
    out_shape=[jax.ShapeDtypeStruct((N, D), jnp.float32),
               jax.ShapeDtypeStruct((N, 64), jnp.float32)],
)

_prescale = pl.pallas_call(
    _prescale_body,
    grid=(_G,),
    in_specs=[_row_spec(D), _row_spec(1), _row_spec(1)],
    out_specs=_row_spec(D),
    out_shape=jax.ShapeDtypeStruct((N, D), jnp.float32),
)

_combine = pl.pallas_call(
    _combine_body,
    grid=(_G,),
    in_specs=[_row_spec(D), _row_spec(D), _row_spec(D), _row_spec(1),
              _row_spec(1), _full_spec(1, D), _full_spec(D, D)],
    out_specs=_row_spec(D),
    out_shape=jax.ShapeDtypeStruct((N, D), jnp.float32),
)

_final = pl.pallas_call(
    _final_body,
    grid=(_G,),
    in_specs=[_row_spec(D), _row_spec(D), _row_spec(D), _row_spec(1),
              _row_spec(1), _full_spec(1, D), _full_spec(D, 64),
              _row_spec(64), _full_spec(1, 64)],
    out_specs=_row_spec(64),
    out_shape=jax.ShapeDtypeStruct((N, 64), jnp.float32),
)


def _pad_edges(src, dst):
    npad = E_WP - E_W
    w = jnp.arange(NW, dtype=jnp.int32).reshape(NW, 1)
    k = jnp.arange(npad, dtype=jnp.int32).reshape(1, npad)
    pad_src = (k * 41 + w * 13) % N
    pad_dst = N + (k + w * 7) % npad
    srcp = jnp.concatenate([src.reshape(NW, E_W), pad_src], axis=1)
    dstp = jnp.concatenate([dst.reshape(NW, E_W), pad_dst], axis=1)
    return srcp.reshape(NW, 1, E_WP), dstp.reshape(NW, NCHP, CHP)


def kernel(x, edge_index, W1, b1, W2, b2, Wc, bc):
    src = edge_index[0].astype(jnp.int32)
    dst = edge_index[1].astype(jnp.int32)
    srcp, dstp = _pad_edges(src, dst)

    degp = _deg_partials(dstp)
    h1, xc = _mm2(x, W1, Wc[D:])
    d0 = degp[0, 0, :N].reshape(N, 1)
    d1 = degp[1, 0, :N].reshape(N, 1)

    hs1 = _prescale(h1, d0, d1)
    p1 = _agg_partials(hs1, srcp, dstp)
    hs2 = _combine(p1[0], p1[1], hs1, d0, d1,
                   b1.reshape(1, D), W2)
    p2 = _agg_partials(hs2, srcp, dstp)
    out = _final(p2[0], p2[1], hs2, d0, d1,
                 b2.reshape(1, D), Wc[:D], xc,
                 bc.reshape(1, 64))
    return out


# speedup vs baseline: 27.0391x; 1.0181x over previous
import functools

import jax
import jax.numpy as jnp
from jax import lax
from jax.experimental import pallas as pl
from jax.experimental.pallas import tpu as pltpu
from jax.experimental.pallas import tpu_sc as plsc

N = 10000
D = 128
E = 320000
NC = 2
NS = 16
NW = NC * NS
E_W = E // NW
N_PAD = 10240
ROWS_W = N_PAD // NS
E_WP = 10240
CHP = 128
NCHP = E_WP // CHP
BLK_CH = 16
E_BLK = BLK_CH * CHP
NBLK = E_WP // E_BLK

_mesh = plsc.VectorSubcoreMesh(core_axis_name="c", subcore_axis_name="s")


@functools.partial(
    pl.kernel,
    out_type=jax.ShapeDtypeStruct((NC, 1, N_PAD), jnp.float32),
    mesh=_mesh,
    scratch_types=[
        pltpu.VMEM((NCHP, CHP), jnp.int32),
        pltpu.VMEM((CHP,), jnp.float32),
        pltpu.VMEM((ROWS_W,), jnp.float32),
        pltpu.VMEM_SHARED((N_PAD,), jnp.float32),
    ],
)
def _deg_partials(dst_hbm, deg_out, didx, onesv, zb, acc):
    c = lax.axis_index("c")
    s = lax.axis_index("s")
    wid = s * NC + c
    for i in range(CHP // 16):
        onesv[pl.ds(i * 16, 16)] = jnp.full((16,), 1.0, jnp.float32)
    for i in range(ROWS_W // 16):
        zb[pl.ds(i * 16, 16)] = jnp.zeros((16,), jnp.float32)
    pltpu.sync_copy(dst_hbm.at[wid], didx)
    pltpu.sync_copy(zb, acc.at[pl.ds(s * ROWS_W, ROWS_W)])
    plsc.subcore_barrier()

    def ebody(i, carry):
        pltpu.sync_copy(onesv, acc.at[didx.at[i]], add=True)
        return carry

    lax.fori_loop(0, NCHP, ebody, 0)
    plsc.subcore_barrier()
    pltpu.sync_copy(acc.at[pl.ds(s * ROWS_W, ROWS_W)],
                    deg_out.at[c, 0, pl.ds(s * ROWS_W, ROWS_W)])


@functools.partial(
    pl.kernel,
    out_type=jax.ShapeDtypeStruct((NC, N_PAD, D), jnp.float32),
    mesh=_mesh,
    scratch_types=[
        pltpu.VMEM((E_BLK,), jnp.int32),
        pltpu.VMEM((BLK_CH, CHP), jnp.int32),
        pltpu.VMEM((CHP, D), jnp.float32),
        pltpu.VMEM((CHP, D), jnp.float32),
        pltpu.VMEM_SHARED((N_PAD, D), jnp.float32),
        pltpu.SemaphoreType.DMA,
        pltpu.SemaphoreType.DMA,
        pltpu.SemaphoreType.DMA,
        pltpu.SemaphoreType.DMA,
    ],
)
def _agg_partials(hs_hbm, src_hbm, dst_hbm, out_hbm,
                  sidx, didx, rows0, rows1, acc, gsem0, gsem1, ssem0, ssem1):
    c = lax.axis_index("c")
    s = lax.axis_index("s")
    wid = s * NC + c
    rbase = s * ROWS_W
    rows = (rows0, rows1)
    gsem = (gsem0, gsem1)
    ssem = (ssem0, ssem1)

    for i in range(32):
        for j in range(D // 16):
            rows1[i, pl.ds(j * 16, 16)] = jnp.zeros((16,), jnp.float32)

    def zbody(i, carry):
        pltpu.sync_copy(rows1.at[pl.ds(0, 32)], acc.at[pl.ds(rbase + i * 32, 32)])
        return carry

    lax.fori_loop(0, ROWS_W // 32, zbody, 0)
    plsc.subcore_barrier()

    def blk_body(blk, carry):
        pltpu.sync_copy(src_hbm.at[wid, 0, pl.ds(blk * E_BLK, E_BLK)], sidx)
        pltpu.sync_copy(dst_hbm.at[wid, pl.ds(blk * BLK_CH, BLK_CH)], didx)
        for k in range(BLK_CH):
            b = k % 2
            if k >= 2:
                pltpu.make_async_copy(rows[b], acc.at[didx.at[k - 2]],
                                      ssem[b]).wait()
            pltpu.async_copy(hs_hbm.at[sidx.at[pl.ds(k * CHP, CHP)]],
                             rows[b], gsem[b])
            if k >= 1:
                pb = (k - 1) % 2
                pltpu.make_async_copy(hs_hbm.at[sidx.at[pl.ds((k - 1) * CHP, CHP)]],
                                      rows[pb], gsem[pb]).wait()
                pltpu.async_copy(rows[pb], acc.at[didx.at[k - 1]], ssem[pb],
                                 add=True)
        lastb = (BLK_CH - 1) % 2
        pltpu.make_async_copy(hs_hbm.at[sidx.at[pl.ds(0, CHP)]],
                              rows[lastb], gsem[lastb]).wait()
        pltpu.async_copy(rows[lastb], acc.at[didx.at[BLK_CH - 1]], ssem[lastb],
                         add=True)
        pltpu.make_async_copy(rows0, acc.at[didx.at[0]], ssem0).wait()
        pltpu.make_async_copy(rows1, acc.at[didx.at[0]], ssem1).wait()
        return carry

    lax.fori_loop(0, NBLK, blk_body, 0)
    plsc.subcore_barrier()
    pltpu.sync_copy(acc.at[pl.ds(rbase, ROWS_W)],
                    out_hbm.at[c, pl.ds(rbase, ROWS_W)])


_BLK = 2000


def _mm2_body(x_ref, wa_ref, wb_ref, oa_ref, ob_ref):
    x = x_ref[...]
    oa_ref[...] = jnp.dot(x, wa_ref[...], preferred_element_type=jnp.float32)
    ob_ref[...] = jnp.dot(x, wb_ref[...], preferred_element_type=jnp.float32)


def _prescale_body(h_ref, d0_ref, d1_ref, o_ref):
    dinv = lax.rsqrt(d0_ref[...] + d1_ref[...] + 1.0)
    o_ref[...] = h_ref[...] * dinv


def _combine_body(p0_ref, p1_ref, hs_ref, d0_ref, d1_ref, b_ref, w_ref, o_ref):
    dinv = lax.rsqrt(d0_ref[...] + d1_ref[...] + 1.0)
    z = dinv * (p0_ref[...] + p1_ref[...] + hs_ref[...]) + b_ref[...]
    z = jnp.maximum(z, 0.0)
    o_ref[...] = jnp.dot(z, w_ref[...], preferred_element_type=jnp.float32) * dinv


def _final_body(p0_ref, p1_ref, hs_ref, d0_ref, d1_ref, b_ref, w_ref, xc_ref,
                bc_ref, o_ref):
    dinv = lax.rsqrt(d0_ref[...] + d1_ref[...] + 1.0)
    z = dinv * (p0_ref[...] + p1_ref[...] + hs_ref[...]) + b_ref[...]
    z = jnp.maximum(z, 0.0)
    o_ref[...] = (jnp.dot(z, w_ref[...], preferred_element_type=jnp.float32)
                  + xc_ref[...] + bc_ref[...])


def _row_spec(w):
    return pl.BlockSpec((_BLK, w), lambda i: (i, 0))


def _full_spec(r, w):
    return pl.BlockSpec((r, w), lambda i: (0, 0))


_G = N // _BLK

_mm2 = pl.pallas_call(
    _mm2_body,
    grid=(_G,),
    in_specs=[_row_spec(D), _full_spec(D, D), _full_spec(D, 64)],
    out_specs=[_row_spec(D), _row_spec(64)],
    out_shape=[jax.ShapeDtypeStruct((N, D), jnp.float32),
               jax.ShapeDtypeStruct((N, 64), jnp.float32)],
)

_prescale = pl.pallas_call(
    _prescale_body,
    grid=(_G,),
    in_specs=[_row_spec(D), _row_spec(1), _row_spec(1)],
    out_specs=_row_spec(D),
    out_shape=jax.ShapeDtypeStruct((N, D), jnp.float32),
)

_combine = pl.pallas_call(
    _combine_body,
    grid=(_G,),
    in_specs=[_row_spec(D), _row_spec(D), _row_spec(D), _row_spec(1),
              _row_spec(1), _full_spec(1, D), _full_spec(D, D)],
    out_specs=_row_spec(D),
    out_shape=jax.ShapeDtypeStruct((N, D), jnp.float32),
)

_final = pl.pallas_call(
    _final_body,
    grid=(_G,),
    in_specs=[_row_spec(D), _row_spec(D), _row_spec(D), _row_spec(1),
              _row_spec(1), _full_spec(1, D), _full_spec(D, 64),
              _row_spec(64), _full_spec(1, 64)],
    out_specs=_row_spec(64),
    out_shape=jax.ShapeDtypeStruct((N, 64), jnp.float32),
)


def _pad_edges(src, dst):
    npad = E_WP - E_W
    w = jnp.arange(NW, dtype=jnp.int32).reshape(NW, 1)
    k = jnp.arange(npad, dtype=jnp.int32).reshape(1, npad)
    pad_src = (k * 41 + w * 13) % N
    pad_dst = N + (k + w * 7) % npad
    srcp = jnp.concatenate([src.reshape(NW, E_W), pad_src], axis=1)
    dstp = jnp.concatenate([dst.reshape(NW, E_W), pad_dst], axis=1)
    return srcp.reshape(NW, 1, E_WP), dstp.reshape(NW, NCHP, CHP)


def kernel(x, edge_index, W1, b1, W2, b2, Wc, bc):
    src = edge_index[0].astype(jnp.int32)
    dst = edge_index[1].astype(jnp.int32)
    srcp, dstp = _pad_edges(src, dst)

    degp = _deg_partials(dstp)
    h1, xc = _mm2(x, W1, Wc[D:])
    d0 = degp[0, 0, :N].reshape(N, 1)
    d1 = degp[1, 0, :N].reshape(N, 1)

    hs1 = _prescale(h1, d0, d1)
    p1 = _agg_partials(hs1, srcp, dstp)
    hs2 = _combine(p1[0], p1[1], hs1, d0, d1,
                   b1.reshape(1, D), W2)
    p2 = _agg_partials(hs2, srcp, dstp)
    out = _final(p2[0], p2[1], hs2, d0, d1,
                 b2.reshape(1, D), Wc[:D], xc,
                 bc.reshape(1, 64))
    return out


# speedup vs baseline: 28.0583x; 1.0377x over previous
import functools

import jax
import jax.numpy as jnp
from jax import lax
from jax.experimental import pallas as pl
from jax.experimental.pallas import tpu as pltpu
from jax.experimental.pallas import tpu_sc as plsc

N = 10000
D = 128
E = 320000
NC = 2
NS = 16
NW = NC * NS
E_W = E // NW
N_PAD = 10240
ROWS_W = N_PAD // NS
E_WP = 10240
CHP = 128
NCHP = E_WP // CHP
BLK_CH = 16
E_BLK = BLK_CH * CHP
NBLK = E_WP // E_BLK

_mesh = plsc.VectorSubcoreMesh(core_axis_name="c", subcore_axis_name="s")


@functools.partial(
    pl.kernel,
    out_type=jax.ShapeDtypeStruct((NC, 1, N_PAD), jnp.float32),
    mesh=_mesh,
    scratch_types=[
        pltpu.VMEM((NCHP, CHP), jnp.int32),
        pltpu.VMEM((CHP,), jnp.float32),
        pltpu.VMEM((ROWS_W,), jnp.float32),
        pltpu.VMEM_SHARED((N_PAD,), jnp.float32),
    ],
)
def _deg_partials(dst_hbm, deg_out, didx, onesv, zb, acc):
    c = lax.axis_index("c")
    s = lax.axis_index("s")
    wid = s * NC + c
    for i in range(CHP // 16):
        onesv[pl.ds(i * 16, 16)] = jnp.full((16,), 1.0, jnp.float32)
    for i in range(ROWS_W // 16):
        zb[pl.ds(i * 16, 16)] = jnp.zeros((16,), jnp.float32)
    pltpu.sync_copy(dst_hbm.at[wid], didx)
    pltpu.sync_copy(zb, acc.at[pl.ds(s * ROWS_W, ROWS_W)])
    plsc.subcore_barrier()

    def ebody(i, carry):
        pltpu.sync_copy(onesv, acc.at[didx.at[i]], add=True)
        return carry

    lax.fori_loop(0, NCHP, ebody, 0)
    plsc.subcore_barrier()
    pltpu.sync_copy(acc.at[pl.ds(s * ROWS_W, ROWS_W)],
                    deg_out.at[c, 0, pl.ds(s * ROWS_W, ROWS_W)])


@functools.partial(
    pl.kernel,
    out_type=jax.ShapeDtypeStruct((NC, N_PAD, D), jnp.float32),
    mesh=_mesh,
    scratch_types=[
        pltpu.VMEM((BLK_CH, CHP), jnp.int32),
        pltpu.VMEM((BLK_CH, CHP), jnp.int32),
        pltpu.VMEM((CHP, D), jnp.float32),
        pltpu.VMEM((CHP, D), jnp.float32),
        pltpu.VMEM_SHARED((N_PAD, D), jnp.float32),
        pltpu.SemaphoreType.DMA,
        pltpu.SemaphoreType.DMA,
    ],
)
def _agg_partials(hs_hbm, src_hbm, dst_hbm, out_hbm,
                  sidx, didx, rows0, rows1, acc, gsem0, gsem1):
    c = lax.axis_index("c")
    s = lax.axis_index("s")
    wid = s * NC + c
    rbase = s * ROWS_W
    rows = (rows0, rows1)
    gsem = (gsem0, gsem1)

    for i in range(32):
        for j in range(D // 16):
            rows1[i, pl.ds(j * 16, 16)] = jnp.zeros((16,), jnp.float32)

    def zbody(i, carry):
        pltpu.sync_copy(rows1.at[pl.ds(0, 32)], acc.at[pl.ds(rbase + i * 32, 32)])
        return carry

    lax.fori_loop(0, ROWS_W // 32, zbody, 0)
    plsc.subcore_barrier()

    def blk_body(blk, carry):
        pltpu.sync_copy(src_hbm.at[wid, pl.ds(blk * BLK_CH, BLK_CH)], sidx)
        pltpu.sync_copy(dst_hbm.at[wid, pl.ds(blk * BLK_CH, BLK_CH)], didx)
        pltpu.async_copy(hs_hbm.at[sidx.at[0]], rows0, gsem0)
        for k in range(1, BLK_CH + 1):
            b = k % 2
            pb = (k - 1) % 2
            if k < BLK_CH:
                pltpu.async_copy(hs_hbm.at[sidx.at[k]], rows[b], gsem[b])
            pltpu.make_async_copy(hs_hbm.at[pl.ds(0, CHP)],
                                  rows[pb], gsem[pb]).wait()
            pltpu.sync_copy(rows[pb], acc.at[didx.at[k - 1]], add=True)
        return carry

    lax.fori_loop(0, NBLK, blk_body, 0)
    plsc.subcore_barrier()
    pltpu.sync_copy(acc.at[pl.ds(rbase, ROWS_W)],
                    out_hbm.at[c, pl.ds(rbase, ROWS_W)])


_BLK = 2000


def _mm2p_body(x_ref, wa_ref, wb_ref, d0_ref, d1_ref, oa_ref, ob_ref):
    x = x_ref[...]
    dinv = lax.rsqrt(d0_ref[...] + d1_ref[...] + 1.0)
    oa_ref[...] = jnp.dot(x, wa_ref[...], preferred_element_type=jnp.float32) * dinv
    ob_ref[...] = jnp.dot(x, wb_ref[...], preferred_element_type=jnp.float32)


def _combine_body(p_ref, hs_ref, d0_ref, d1_ref, b_ref, w_ref, o_ref):
    dinv = lax.rsqrt(d0_ref[...] + d1_ref[...] + 1.0)
    z = dinv * (p_ref[0] + p_ref[1] + hs_ref[...]) + b_ref[...]
    z = jnp.maximum(z, 0.0)
    o_ref[...] = jnp.dot(z, w_ref[...], preferred_element_type=jnp.float32) * dinv


def _final_body(p_ref, hs_ref, d0_ref, d1_ref, b_ref, w_ref, xc_ref,
                bc_ref, o_ref):
    dinv = lax.rsqrt(d0_ref[...] + d1_ref[...] + 1.0)
    z = dinv * (p_ref[0] + p_ref[1] + hs_ref[...]) + b_ref[...]
    z = jnp.maximum(z, 0.0)
    o_ref[...] = (jnp.dot(z, w_ref[...], preferred_element_type=jnp.float32)
                  + xc_ref[...] + bc_ref[...])


def _row_spec(w):
    return pl.BlockSpec((_BLK, w), lambda i: (i, 0))


def _full_spec(r, w):
    return pl.BlockSpec((r, w), lambda i: (0, 0))


_part_spec = pl.BlockSpec((2, _BLK, D), lambda i: (0, i, 0))

_G = N // _BLK

_mm2p = pl.pallas_call(
    _mm2p_body,
    grid=(_G,),
    in_specs=[_row_spec(D), _full_spec(D, D), _full_spec(D, 64),
              _row_spec(1), _row_spec(1)],
    out_specs=[_row_spec(D), _row_spec(64)],
    out_shape=[jax.ShapeDtypeStruct((N, D), jnp.float32),
               jax.ShapeDtypeStruct((N, 64), jnp.float32)],
)

_combine = pl.pallas_call(
    _combine_body,
    grid=(_G,),
    in_specs=[_part_spec, _row_spec(D), _row_spec(1),
              _row_spec(1), _full_spec(1, D), _full_spec(D, D)],
    out_specs=_row_spec(D),
    out_shape=jax.ShapeDtypeStruct((N, D), jnp.float32),
)

_final = pl.pallas_call(
    _final_body,
    grid=(_G,),
    in_specs=[_part_spec, _row_spec(D), _row_spec(1),
              _row_spec(1), _full_spec(1, D), _full_spec(D, 64),
              _row_spec(64), _full_spec(1, 64)],
    out_specs=_row_spec(64),
    out_shape=jax.ShapeDtypeStruct((N, 64), jnp.float32),
)


def _pad_edges(src, dst):
    npad = E_WP - E_W
    w = jnp.arange(NW, dtype=jnp.int32).reshape(NW, 1)
    k = jnp.arange(npad, dtype=jnp.int32).reshape(1, npad)
    pad_src = (k * 41 + w * 13) % N
    pad_dst = N + (k + w * 7) % npad
    srcp = jnp.concatenate([src.reshape(NW, E_W), pad_src], axis=1)
    dstp = jnp.concatenate([dst.reshape(NW, E_W), pad_dst], axis=1)
    return srcp.reshape(NW, NCHP, CHP), dstp.reshape(NW, NCHP, CHP)


def kernel(x, edge_index, W1, b1, W2, b2, Wc, bc):
    src = edge_index[0].astype(jnp.int32)
    dst = edge_index[1].astype(jnp.int32)
    srcp, dstp = _pad_edges(src, dst)

    degp = _deg_partials(dstp)
    d0 = degp[0, 0, :N].reshape(N, 1)
    d1 = degp[1, 0, :N].reshape(N, 1)

    hs1, xc = _mm2p(x, W1, Wc[D:], d0, d1)
    p1 = _agg_partials(hs1, srcp, dstp)
    hs2 = _combine(p1, hs1, d0, d1,
                   b1.reshape(1, D), W2)
    p2 = _agg_partials(hs2, srcp, dstp)
    out = _final(p2, hs2, d0, d1,
                 b2.reshape(1, D), Wc[:D], xc,
                 bc.reshape(1, 64))
    return out
